# baseline (speedup 1.0000x reference)
import functools

import jax
import jax.numpy as jnp
from jax import lax
from jax.experimental import pallas as pl
from jax.experimental.pallas import tpu as pltpu
from jax.experimental.pallas import tpu_sc as plsc

_H = 256
_NW = 32


@functools.lru_cache(maxsize=None)
def _make_gather(M, NT, D):
    per = M // _NW
    CK = 96
    NS = 4
    n_full = per // CK
    tail = per - n_full * CK
    n_q = n_full // NS
    assert n_full % NS == 0 and tail % 8 == 0, (per, CK, tail)
    mesh = plsc.VectorSubcoreMesh(core_axis_name="c", subcore_axis_name="s")

    @functools.partial(
        pl.kernel,
        mesh=mesh,
        out_type=jax.ShapeDtypeStruct((M, D), jnp.int32),
        scratch_types=[pltpu.VMEM((per,), jnp.int32)]
        + [pltpu.VMEM((CK, D), jnp.int32) for _ in range(NS)]
        + [pltpu.SemaphoreType.DMA for _ in range(NS + 1)],
    )
    def gk(t_hbm, i_hbm, o_hbm, i_v, *bufs_sems):
        bufs = bufs_sems[0:NS]
        gs = bufs_sems[NS:2 * NS]
        ws = bufs_sems[2 * NS]
        wid = lax.axis_index("s") * 2 + lax.axis_index("c")
        base = wid * per
        pltpu.sync_copy(i_hbm.at[pl.ds(base, per)], i_v)

        def issue_g(off, k):
            pltpu.async_copy(
                t_hbm.at[i_v.at[pl.ds(off, CK)]], bufs[k], gs[k])

        for k in range(NS):
            issue_g(k * CK, k)

        def body(q, _):
            for k in range(NS):
                off = (q * NS + k) * CK
                pltpu.make_async_copy(
                    t_hbm.at[i_v.at[pl.ds(off, CK)]], bufs[k],
                    gs[k]).wait()
                pltpu.async_copy(
                    bufs[k], o_hbm.at[pl.ds(base + off, CK)], ws).wait()
                pl.when(q < n_q - 1)(
                    lambda off=off, k=k: issue_g(off + NS * CK, k))
            return 0

        lax.fori_loop(0, n_q, body, 0)

        if tail:
            toff = n_full * CK
            pltpu.async_copy(
                t_hbm.at[i_v.at[pl.ds(toff, tail)]],
                bufs[0].at[pl.ds(0, tail)], gs[0]).wait()
            pltpu.async_copy(
                bufs[0].at[pl.ds(0, tail)],
                o_hbm.at[pl.ds(base + toff, tail)], ws).wait()

    return gk


def _sc_gather_pair(A, B, e0, e1):
    E = e0.shape[0]
    N, D2 = A.shape
    D = D2 // 2

    def pack(x):
        xb = x.astype(jnp.bfloat16).reshape(N, D, 2)
        return lax.bitcast_convert_type(xb, jnp.int32)

    tab = jnp.concatenate([pack(A), pack(B)], axis=0)
    idx = jnp.concatenate([e0, e1 + N])
    out = _make_gather(2 * E, 2 * N, D)(tab, idx)
    ab = lax.bitcast_convert_type(out, jnp.bfloat16).reshape(2, E, D2)
    return ab[0], ab[1]


def _lin_kernel(x_ref, w_ref, b_ref, o_ref):
    o_ref[...] = (
        jnp.dot(x_ref[...], w_ref[...], preferred_element_type=jnp.float32)
        + b_ref[...]
    )


def _plin(x, W, b, block=1000):
    N, din = x.shape
    dout = W.shape[0]
    assert N % block == 0, (N, block)
    Wt = W.T
    return pl.pallas_call(
        _lin_kernel,
        grid=(N // block,),
        in_specs=[
            pl.BlockSpec((block, din), lambda i: (i, 0)),
            pl.BlockSpec((din, dout), lambda i: (0, 0)),
            pl.BlockSpec((dout,), lambda i: (0,)),
        ],
        out_specs=pl.BlockSpec((block, dout), lambda i: (i, 0)),
        out_shape=jax.ShapeDtypeStruct((N, dout), jnp.float32),
    )(x, Wt, b)


def _res_kernel(x_ref, w1_ref, b1_ref, w2_ref, b2_ref, o_ref):
    h = (
        jnp.dot(x_ref[...], w1_ref[...], preferred_element_type=jnp.float32)
        + b1_ref[...]
    )
    o_ref[...] = (
        jnp.dot(h, w2_ref[...], preferred_element_type=jnp.float32)
        + b2_ref[...]
        + x_ref[...]
    )


def _pres(p, x, block=1000):
    N, d = x.shape
    W1, b1 = p["l1"]
    W2, b2 = p["l2"]
    return pl.pallas_call(
        _res_kernel,
        grid=(N // block,),
        in_specs=[
            pl.BlockSpec((block, d), lambda i: (i, 0)),
            pl.BlockSpec((d, d), lambda i: (0, 0)),
            pl.BlockSpec((d,), lambda i: (0,)),
            pl.BlockSpec((d, d), lambda i: (0, 0)),
            pl.BlockSpec((d,), lambda i: (0,)),
        ],
        out_specs=pl.BlockSpec((block, d), lambda i: (i, 0)),
        out_shape=jax.ShapeDtypeStruct((N, d), jnp.float32),
    )(x, W1.T, b1, W2.T, b2)


@functools.lru_cache(maxsize=None)
def _make_segment_reduce(E, D, op):
    BS = 320
    assert D % 16 == 0
    CH = 4000
    assert E % CH == 0
    n_ch = E // CH
    CAP = 16384
    GC = 64
    is_max = op == "max"
    init = float("-inf") if is_max else 0.0
    pad_entry = (BS << 18) | (E - 1)
    mesh = plsc.VectorSubcoreMesh(core_axis_name="c", subcore_axis_name="s")

    @functools.partial(
        pl.kernel,
        mesh=mesh,
        out_type=jax.ShapeDtypeStruct((_NW, BS, D), jnp.float32),
        scratch_types=[
            pltpu.VMEM((CH,), jnp.int32),
            pltpu.VMEM((CAP + 32,), jnp.int32),
            pltpu.VMEM((GC,), jnp.int32),
            pltpu.VMEM((GC, D), jnp.float32),
            pltpu.VMEM((BS + 1, D), jnp.float32),
            pltpu.SemaphoreType.DMA,
        ],
    )
    def rk(pay_hbm, dst_hbm, out_hbm, dv, lst, gi, pbuf, acc, sem):
        wid = lax.axis_index("s") * 2 + lax.axis_index("c")
        lo = wid * BS
        ivec = lax.iota(jnp.int32, 16)
        fill = jnp.full((16,), init, jnp.float32)
        padv = jnp.full((16,), pad_entry, jnp.int32)

        def init_body(i, _):
            for r in range(D // 16):
                acc[i, pl.ds(r * 16, 16)] = fill
            return 0

        lax.fori_loop(0, BS, init_body, 0)

        def drain(n):
            lst[pl.ds(n, 16)] = padv
            n16 = ((n + 15) // 16) * 16

            def dbody(ci, _):
                j0 = ci * GC
                for sl in range(GC // 16):
                    v = lst[pl.ds(j0 + sl * 16, 16)]
                    gi[pl.ds(sl * 16, 16)] = jnp.minimum(
                        v & 0x3FFFF, E - 1)
                pltpu.async_copy(pay_hbm.at[gi], pbuf, sem).wait()
                ng = jnp.minimum(GC, n16 - j0) // 16

                def gbody(g, _):
                    gv = lst[pl.ds(j0 + g * 16, 16)]
                    for l in range(16):
                        local = gv[l] >> 18
                        row = g * 16 + l
                        for r in range(D // 16):
                            x = pbuf[row, pl.ds(r * 16, 16)]
                            if is_max:
                                cur = acc[local, pl.ds(r * 16, 16)]
                                acc[local, pl.ds(r * 16, 16)] = (
                                    jnp.maximum(cur, x))
                            else:
                                plsc.addupdate(
                                    acc.at[local, pl.ds(r * 16, 16)], x)
                    return 0

                lax.fori_loop(0, ng, gbody, 0)
                return 0

            lax.fori_loop(0, (n16 + GC - 1) // GC, dbody, 0)

        def chunk_body(c, off):
            pltpu.sync_copy(dst_hbm.at[pl.ds(c * CH, CH)], dv)

            def scan_body(v, off):
                d16 = dv[pl.ds(v * 16, 16)]
                u = d16 - lo
                msk = (u >= 0) & (u < BS)
                eid = ivec + (c * CH + v * 16)
                packed = (u << 18) | eid
                plsc.store_compressed(lst.at[pl.ds(off, 16)], packed,
                                      mask=msk)
                pc = plsc.all_reduce_population_count(msk)
                return off + pc[0]

            off = lax.fori_loop(0, CH // 16, scan_body, off)
            pl.when(off >= CAP - CH)(lambda: drain(off))
            return jnp.where(off >= CAP - CH, 0, off)

        off = lax.fori_loop(0, n_ch, chunk_body, 0)
        pl.when(off > 0)(lambda: drain(off))

        if is_max:
            ninf = jnp.float32(float("-inf"))

            def fix_body(i, _):
                for r in range(D // 16):
                    v = acc[i, pl.ds(r * 16, 16)]
                    acc[i, pl.ds(r * 16, 16)] = jnp.where(
                        v == ninf, 0.0, v)
                return 0

            lax.fori_loop(0, BS, fix_body, 0)

        pltpu.sync_copy(acc.at[pl.ds(0, BS)], out_hbm.at[wid])

    return rk


def _sc_segment_reduce(payload, dst, n_out, op):
    E, D = payload.shape
    out = _make_segment_reduce(E, D, op)(payload, dst)
    return out.reshape(320 * _NW, D)[:n_out]


def _edge_kernel(a_ref, b_ref, wk_ref, w2t_ref, bk_ref, b2_ref,
                 f1_ref, f2_ref):
    H = _H
    u = a_ref[...].astype(jnp.float32) + b_ref[...].astype(jnp.float32)
    h = jnp.where(u >= 0.0, u, 0.2 * u)
    logit = jnp.sum(h * wk_ref[...], axis=1, keepdims=True) + bk_ref[...]
    k = jax.nn.sigmoid(logit)
    m2 = (
        jnp.dot(h.astype(jnp.bfloat16), w2t_ref[...],
                preferred_element_type=jnp.float32)
        + b2_ref[...]
    )
    f = m2 * k
    f1_ref[...] = f[:, :H].astype(jnp.bfloat16)
    f2_ref[...] = f[:, H:].astype(jnp.bfloat16)


def _pedge(Ag, Bg, wk, bk, W2r, b2r, block=2000):
    E, d2 = Ag.shape
    H = _H
    return pl.pallas_call(
        _edge_kernel,
        grid=(E // block,),
        in_specs=[
            pl.BlockSpec((block, d2), lambda i: (i, 0)),
            pl.BlockSpec((block, d2), lambda i: (i, 0)),
            pl.BlockSpec((1, d2), lambda i: (0, 0)),
            pl.BlockSpec((d2, d2), lambda i: (0, 0)),
            pl.BlockSpec((1, 1), lambda i: (0, 0)),
            pl.BlockSpec((d2,), lambda i: (0,)),
        ],
        out_specs=[
            pl.BlockSpec((block, H), lambda i: (i, 0)),
            pl.BlockSpec((block, H), lambda i: (i, 0)),
        ],
        out_shape=[
            jax.ShapeDtypeStruct((E, H), jnp.bfloat16),
            jax.ShapeDtypeStruct((E, H), jnp.bfloat16),
        ],
    )(Ag, Bg, wk.reshape(1, d2), W2r.T.astype(jnp.bfloat16),
      bk.reshape(1, 1), b2r)


def _mp_direction(x_src, x_dst, edge, msg, red, G, postCat, x_in1, n_dst):
    H = _H
    W1, b1 = msg["l1"]
    W2, b2 = msg["l2"]
    A = _plin(x_src, W1[:, :H], jnp.zeros((2 * H,), jnp.float32))
    B = _plin(x_dst, W1[:, H:], b1)
    Ag, Bg = _sc_gather_pair(A, B, edge[0], edge[1])
    wk = W2[0]
    bk = b2[0:1]
    f1, f2 = _pedge(Ag, Bg, wk, bk, W2[1:], b2[1:])
    nf1 = jax.ops.segment_sum(f1, edge[1], num_segments=n_dst)
    m = jax.ops.segment_max(f2, edge[1], num_segments=n_dst)
    nf2 = jnp.where(jnp.isneginf(m), 0.0, m)
    nf1 = nf1.astype(jnp.float32)
    nf2 = nf2.astype(jnp.float32)
    cat = jnp.concatenate([x_dst, nf1, nf2], axis=1)
    new_x = _plin(cat, red[0], red[1])
    new_x = _plin(new_x, G[0], G[1])
    cat2 = jnp.concatenate([new_x, x_in1], axis=1)
    return x_dst + _plin(cat2, postCat[0], postCat[1])


def kernel(nf_gc, nf_gn, nf_gc_in1, nf_gn_in1, edge_c2n, edge_n2c, params):
    p = params
    x_gc_in1 = _plin(nf_gc_in1, p["gc_in1"][0], p["gc_in1"][1])
    x_gn_in1 = _plin(nf_gn_in1, p["gn_in1"][0], p["gn_in1"][1])
    x_gc = _pres(p["res_gc_1"], nf_gc)
    x_gn = _pres(p["res_gn_1"], nf_gn)
    NN = nf_gn.shape[0]
    NC = nf_gc.shape[0]
    x_gn = _mp_direction(
        x_gc, x_gn, edge_c2n, p["msg_c2n"], p["red_c2n"], p["Gcn"],
        p["postCatGcn"], x_gn_in1, NN,
    )
    x_gn = _pres(p["res_gn_2"], x_gn)
    x_gc = _pres(p["res_gc_2"], x_gc)
    x_gc = _mp_direction(
        x_gn, x_gc, edge_n2c, p["msg_n2c"], p["red_n2c"], p["Gnc"],
        p["postCatGnc"], x_gc_in1, NC,
    )
    return (x_gc, x_gn)


# speedup vs baseline: 2.7470x; 2.7470x over previous
import jax
import jax.numpy as jnp
from jax.experimental import pallas as pl

_H = 256


def _lin_kernel(x_ref, w_ref, b_ref, o_ref):
    o_ref[...] = (
        jnp.dot(x_ref[...], w_ref[...], preferred_element_type=jnp.float32)
        + b_ref[...]
    )


def _plin(x, W, b, block=1000):
    N, din = x.shape
    dout = W.shape[0]
    assert N % block == 0, (N, block)
    return pl.pallas_call(
        _lin_kernel,
        grid=(N // block,),
        in_specs=[
            pl.BlockSpec((block, din), lambda i: (i, 0)),
            pl.BlockSpec((din, dout), lambda i: (0, 0)),
            pl.BlockSpec((dout,), lambda i: (0,)),
        ],
        out_specs=pl.BlockSpec((block, dout), lambda i: (i, 0)),
        out_shape=jax.ShapeDtypeStruct((N, dout), jnp.float32),
    )(x, W.T, b)


def _res_kernel(x_ref, w1_ref, b1_ref, w2_ref, b2_ref, o_ref):
    h = (
        jnp.dot(x_ref[...], w1_ref[...], preferred_element_type=jnp.float32)
        + b1_ref[...]
    )
    o_ref[...] = (
        jnp.dot(h, w2_ref[...], preferred_element_type=jnp.float32)
        + b2_ref[...]
        + x_ref[...]
    )


def _pres(p, x, block=1000):
    N, d = x.shape
    W1, b1 = p["l1"]
    W2, b2 = p["l2"]
    return pl.pallas_call(
        _res_kernel,
        grid=(N // block,),
        in_specs=[
            pl.BlockSpec((block, d), lambda i: (i, 0)),
            pl.BlockSpec((d, d), lambda i: (0, 0)),
            pl.BlockSpec((d,), lambda i: (0,)),
            pl.BlockSpec((d, d), lambda i: (0, 0)),
            pl.BlockSpec((d,), lambda i: (0,)),
        ],
        out_specs=pl.BlockSpec((block, d), lambda i: (i, 0)),
        out_shape=jax.ShapeDtypeStruct((N, d), jnp.float32),
    )(x, W1.T, b1, W2.T, b2)


def _edge_kernel(a_ref, b_ref, wk_ref, w2t_ref, bk_ref, b2_ref,
                 f1_ref, f2_ref):
    H = _H
    u = a_ref[...].astype(jnp.float32) + b_ref[...].astype(jnp.float32)
    h = jnp.where(u >= 0.0, u, 0.2 * u)
    logit = jnp.sum(h * wk_ref[...], axis=1, keepdims=True) + bk_ref[...]
    k = jax.nn.sigmoid(logit)
    m2 = (
        jnp.dot(h.astype(jnp.bfloat16), w2t_ref[...],
                preferred_element_type=jnp.float32)
        + b2_ref[...]
    )
    f = m2 * k
    f1_ref[...] = f[:, :H].astype(jnp.bfloat16)
    f2_ref[...] = f[:, H:].astype(jnp.bfloat16)


def _pedge(Ag, Bg, wk, bk, W2r, b2r, block=2000):
    E, d2 = Ag.shape
    H = _H
    return pl.pallas_call(
        _edge_kernel,
        grid=(E // block,),
        in_specs=[
            pl.BlockSpec((block, d2), lambda i: (i, 0)),
            pl.BlockSpec((block, d2), lambda i: (i, 0)),
            pl.BlockSpec((1, d2), lambda i: (0, 0)),
            pl.BlockSpec((d2, d2), lambda i: (0, 0)),
            pl.BlockSpec((1, 1), lambda i: (0, 0)),
            pl.BlockSpec((d2,), lambda i: (0,)),
        ],
        out_specs=[
            pl.BlockSpec((block, H), lambda i: (i, 0)),
            pl.BlockSpec((block, H), lambda i: (i, 0)),
        ],
        out_shape=[
            jax.ShapeDtypeStruct((E, H), jnp.bfloat16),
            jax.ShapeDtypeStruct((E, H), jnp.bfloat16),
        ],
    )(Ag, Bg, wk.reshape(1, d2), W2r.T.astype(jnp.bfloat16),
      bk.reshape(1, 1), b2r)


def _mp_direction(x_src, x_dst, edge, msg, red, G, postCat, x_in1, n_dst):
    H = _H
    W1, b1 = msg["l1"]
    W2, b2 = msg["l2"]
    A = _plin(x_src, W1[:, :H], jnp.zeros((2 * H,), jnp.float32))
    B = _plin(x_dst, W1[:, H:], b1)
    Ag = A.astype(jnp.bfloat16)[edge[0]]
    Bg = B.astype(jnp.bfloat16)[edge[1]]
    wk = W2[0]
    bk = b2[0:1]
    f1, f2 = _pedge(Ag, Bg, wk, bk, W2[1:], b2[1:])
    nf1 = jax.ops.segment_sum(f1, edge[1], num_segments=n_dst)
    m = jax.ops.segment_max(f2, edge[1], num_segments=n_dst)
    nf2 = jnp.where(jnp.isneginf(m), 0.0, m)
    nf1 = nf1.astype(jnp.float32)
    nf2 = nf2.astype(jnp.float32)
    cat = jnp.concatenate([x_dst, nf1, nf2], axis=1)
    new_x = _plin(cat, red[0], red[1])
    new_x = _plin(new_x, G[0], G[1])
    cat2 = jnp.concatenate([new_x, x_in1], axis=1)
    return x_dst + _plin(cat2, postCat[0], postCat[1])


def kernel(nf_gc, nf_gn, nf_gc_in1, nf_gn_in1, edge_c2n, edge_n2c, params):
    p = params
    x_gc_in1 = _plin(nf_gc_in1, p["gc_in1"][0], p["gc_in1"][1])
    x_gn_in1 = _plin(nf_gn_in1, p["gn_in1"][0], p["gn_in1"][1])
    x_gc = _pres(p["res_gc_1"], nf_gc)
    x_gn = _pres(p["res_gn_1"], nf_gn)
    NN = nf_gn.shape[0]
    NC = nf_gc.shape[0]
    x_gn = _mp_direction(
        x_gc, x_gn, edge_c2n, p["msg_c2n"], p["red_c2n"], p["Gcn"],
        p["postCatGcn"], x_gn_in1, NN,
    )
    x_gn = _pres(p["res_gn_2"], x_gn)
    x_gc = _pres(p["res_gc_2"], x_gc)
    x_gc = _mp_direction(
        x_gn, x_gc, edge_n2c, p["msg_n2c"], p["red_n2c"], p["Gnc"],
        p["postCatGnc"], x_gc_in1, NC,
    )
    return (x_gc, x_gn)
